# baseline (speedup 1.0000x reference)
import functools

import jax
import jax.numpy as jnp
from jax import lax
from jax.experimental import pallas as pl
from jax.experimental.pallas import tpu as pltpu
from jax.experimental.pallas import tpu_sc as plsc

_FIELD_DIM = 100000
_EMBED_DIM = 32
_B = 4096
_F = 26
_N = _B * _F
_NW = 32
_CHUNK = _N // _NW
_LANES = 16

_K = 13
_S = _CHUNK // _K
_SO = _S * _EMBED_DIM

_mesh = plsc.VectorSubcoreMesh(core_axis_name="c", subcore_axis_name="s")


@functools.partial(
    pl.kernel,
    mesh=_mesh,
    out_type=jax.ShapeDtypeStruct((_N * _EMBED_DIM,), jnp.float32),
    scratch_types=[
        pltpu.VMEM((_S,), jnp.int32),
        pltpu.VMEM((_S,), jnp.int32),
        pltpu.VMEM((_S,), jnp.int32),
        pltpu.VMEM((_S,), jnp.int32),
        pltpu.VMEM((_S,), jnp.int32),
        pltpu.VMEM((_S,), jnp.int32),
        pltpu.VMEM((_S, 128), jnp.float32),
        pltpu.VMEM((_S, 128), jnp.float32),
        pltpu.VMEM((_SO,), jnp.float32),
        pltpu.VMEM((_SO,), jnp.float32),
        pltpu.SemaphoreType.DMA((2,)),
        pltpu.SemaphoreType.DMA((2,)),
        pltpu.SemaphoreType.DMA((2,)),
    ],
    compiler_params=pltpu.CompilerParams(needs_layout_passes=False),
)
def _emb_lookup(xt_hbm, tab_hbm, out_hbm, x_v0, x_v1, p_v0, p_v1, w_v0, w_v1,
                rows_v0, rows_v1, comp_v0, comp_v1, xsem, gsem, psem):
    wid = lax.axis_index("s") * 2 + lax.axis_index("c")
    base = wid * _CHUNK
    lane = lax.iota(jnp.int32, _LANES)
    x_v = [x_v0, x_v1]
    p_v = [p_v0, p_v1]
    w_v = [w_v0, w_v1]
    rows_v = [rows_v0, rows_v1]
    comp_v = [comp_v0, comp_v1]

    def start_xload(s):
        return pltpu.async_copy(
            xt_hbm.at[pl.ds(base + s * _S, _S)], x_v[s % 2], xsem.at[s % 2])

    def prep(s):
        b = s % 2

        def body(t, carry):
            sl = pl.ds(t * _LANES, _LANES)
            pos = (base + s * _S) + t * _LANES + lane
            idx = x_v[b][sl] + (pos >> 12) * _FIELD_DIM
            p_v[b][sl] = idx >> 2
            w_v[b][sl] = idx & 3
            return carry

        lax.fori_loop(0, _S // _LANES, body, 0)

    def start_gather(s):
        b = s % 2
        return pltpu.async_copy(tab_hbm.at[p_v[b]], rows_v[b], gsem.at[b])

    def extract(s):
        b = s % 2

        def body(t, carry):
            q0 = t * _LANES
            i = q0 >> 5
            c0 = q0 & 31
            i_vec = lane * 0 + i
            w_vec = plsc.load_gather(w_v[b], [i_vec])
            src = plsc.load_gather(
                rows_v[b], [i_vec, w_vec * 32 + (c0 + lane)])
            comp_v[b][pl.ds(q0, _LANES)] = src
            return carry

        lax.fori_loop(0, _SO // _LANES, body, 0)

    def start_put(s):
        b = s % 2
        return pltpu.async_copy(
            comp_v[b],
            out_hbm.at[pl.ds((base + s * _S) * _EMBED_DIM, _SO)],
            psem.at[b])

    xs = [start_xload(0), start_xload(1)]
    g = [None] * _K
    p = [None] * _K
    xs[0].wait()
    prep(0)
    g[0] = start_gather(0)
    xs[1].wait()
    prep(1)
    g[1] = start_gather(1)
    for s in range(_K):
        g[s].wait()
        extract(s)
        p[s] = start_put(s)
        if s + 2 < _K:
            xs[s % 2] = start_xload(s + 2)
            xs[s % 2].wait()
            prep(s + 2)
            p[s].wait()
            g[s + 2] = start_gather(s + 2)
    p[_K - 2].wait()
    p[_K - 1].wait()


def kernel(x, table):
    xt = x.T.reshape(_N).astype(jnp.int32)
    tab128 = table.reshape(table.shape[0] // 4, 4 * _EMBED_DIM)
    out = _emb_lookup(xt, tab128)
    return out.reshape(_F, _B, _EMBED_DIM).transpose(1, 0, 2)


# speedup vs baseline: 1.0653x; 1.0653x over previous
import functools

import jax
import jax.numpy as jnp
from jax import lax
from jax.experimental import pallas as pl
from jax.experimental.pallas import tpu as pltpu
from jax.experimental.pallas import tpu_sc as plsc

_FIELD_DIM = 100000
_EMBED_DIM = 32
_B = 4096
_F = 26
_N = _B * _F
_NW = 32
_CHUNK = _N // _NW
_LANES = 16

_K = 8
_S = _CHUNK // _K

_mesh = plsc.VectorSubcoreMesh(core_axis_name="c", subcore_axis_name="s")


@functools.partial(
    pl.kernel,
    mesh=_mesh,
    out_type=jax.ShapeDtypeStruct((_N, _EMBED_DIM), jnp.float32),
    scratch_types=[
        pltpu.VMEM((_S,), jnp.int32),
        pltpu.VMEM((_S,), jnp.int32),
        pltpu.VMEM((_S,), jnp.int32),
        pltpu.VMEM((_S,), jnp.int32),
        pltpu.VMEM((_S, _EMBED_DIM), jnp.float32),
        pltpu.VMEM((_S, _EMBED_DIM), jnp.float32),
        pltpu.SemaphoreType.DMA((2,)),
        pltpu.SemaphoreType.DMA((2,)),
        pltpu.SemaphoreType.DMA((2,)),
    ],
    compiler_params=pltpu.CompilerParams(
        use_tc_tiling_on_sc=False, needs_layout_passes=False),
)
def _emb_lookup(xt_hbm, tab_hbm, out_hbm, x_v0, x_v1, p_v0, p_v1,
                rows_v0, rows_v1, xsem, gsem, psem):
    wid = lax.axis_index("s") * 2 + lax.axis_index("c")
    base = wid * _CHUNK
    lane = lax.iota(jnp.int32, _LANES)
    x_v = [x_v0, x_v1]
    p_v = [p_v0, p_v1]
    rows_v = [rows_v0, rows_v1]

    def start_xload(s):
        return pltpu.async_copy(
            xt_hbm.at[pl.ds(base + s * _S, _S)], x_v[s % 2], xsem.at[s % 2])

    def prep(s):
        b = s % 2

        def body(t, carry):
            sl = pl.ds(t * _LANES, _LANES)
            pos = (base + s * _S) + t * _LANES + lane
            p_v[b][sl] = x_v[b][sl] + (pos >> 12) * _FIELD_DIM
            return carry

        lax.fori_loop(0, _S // _LANES, body, 0)

    def start_gather(s):
        b = s % 2
        return pltpu.async_copy(tab_hbm.at[p_v[b]], rows_v[b], gsem.at[b])

    def start_put(s):
        b = s % 2
        return pltpu.async_copy(
            rows_v[b], out_hbm.at[pl.ds(base + s * _S, _S)], psem.at[b])

    xs = [start_xload(0), start_xload(1)]
    g = [None] * _K
    p = [None] * _K
    xs[0].wait()
    prep(0)
    g[0] = start_gather(0)
    xs[1].wait()
    prep(1)
    g[1] = start_gather(1)
    for s in range(_K):
        g[s].wait()
        p[s] = start_put(s)
        if s + 2 < _K:
            xs[s % 2] = start_xload(s + 2)
            xs[s % 2].wait()
            prep(s + 2)
            p[s].wait()
            g[s + 2] = start_gather(s + 2)
    p[_K - 2].wait()
    p[_K - 1].wait()


def kernel(x, table):
    xt = x.T.reshape(_N).astype(jnp.int32)
    out = _emb_lookup(xt, table)
    return out.reshape(_F, _B, _EMBED_DIM).transpose(1, 0, 2)


# speedup vs baseline: 1.6703x; 1.5680x over previous
import functools

import jax
import jax.numpy as jnp
from jax import lax
from jax.experimental import pallas as pl
from jax.experimental.pallas import tpu as pltpu
from jax.experimental.pallas import tpu_sc as plsc

_FIELD_DIM = 100000
_EMBED_DIM = 32
_B = 4096
_F = 26
_N = _B * _F
_NW = 32
_CHUNK = _N // _NW
_LANES = 16

_K = 26
_S = _CHUNK // _K

_mesh = plsc.VectorSubcoreMesh(core_axis_name="c", subcore_axis_name="s")


@functools.partial(
    pl.kernel,
    mesh=_mesh,
    out_type=jax.ShapeDtypeStruct((_N, _EMBED_DIM), jnp.float32),
    scratch_types=[
        pltpu.VMEM((_S,), jnp.int32),
        pltpu.VMEM((_S,), jnp.int32),
        pltpu.VMEM((_S,), jnp.int32),
        pltpu.VMEM((_S,), jnp.int32),
        pltpu.VMEM((_S, _EMBED_DIM), jnp.float32),
        pltpu.VMEM((_S, _EMBED_DIM), jnp.float32),
        pltpu.SemaphoreType.DMA((2,)),
        pltpu.SemaphoreType.DMA((2,)),
        pltpu.SemaphoreType.DMA((2,)),
        pltpu.SemaphoreType.DMA((2,)),
    ],
    compiler_params=pltpu.CompilerParams(needs_layout_passes=False),
)
def _emb_lookup(xt_hbm, tab_hbm, out_hbm, x_v0, x_v1, p_v0, p_v1,
                rows_v0, rows_v1, xsem, ssem, gsem, psem):
    wid = lax.axis_index("s") * 2 + lax.axis_index("c")
    base = wid * _CHUNK
    lane = lax.iota(jnp.int32, _LANES)
    x_v = [x_v0, x_v1]
    p_v = [p_v0, p_v1]
    rows_v = [rows_v0, rows_v1]

    def start_xload(s):
        return pltpu.async_copy(
            xt_hbm.at[pl.ds(base + s * _S, _S)], x_v[s % 2], xsem.at[s % 2])

    def prep(s):
        b = s % 2

        def body(t, carry):
            sl = pl.ds(t * _LANES, _LANES)
            pos = (base + s * _S) + t * _LANES + lane
            p_v[b][sl] = x_v[b][sl] + (pos >> 12) * _FIELD_DIM
            return carry

        lax.fori_loop(0, _S // _LANES, body, 0)

    def gather(s):
        b = s % 2

        def body(i, carry):
            grp = p_v[b][pl.ds((i >> 4) * _LANES, _LANES)]
            r = lax.reduce_max(
                jnp.where(lane == (i & 15), grp, 0), axes=(0,))
            pltpu.async_copy(
                tab_hbm.at[pl.ds(r, 1)], rows_v[b].at[pl.ds(i, 1)],
                gsem.at[b])
            return carry

        lax.fori_loop(0, _S, body, 0)

    def drain_gather(s):
        b = s % 2
        pltpu.make_async_copy(
            tab_hbm.at[pl.ds(0, _S)], rows_v[b], gsem.at[b]).wait()

    def start_put(s):
        b = s % 2
        return pltpu.async_copy(
            rows_v[b], out_hbm.at[pl.ds(base + s * _S, _S)], psem.at[b])

    xs = [start_xload(0), start_xload(1)]
    p = [None] * _K
    xs[0].wait()
    prep(0)
    xs[1].wait()
    prep(1)
    gather(0)
    for s in range(_K):
        drain_gather(s)
        p[s] = start_put(s)
        if s + 1 < _K:
            gather(s + 1)
        if s + 2 < _K:
            xs[s % 2] = start_xload(s + 2)
            xs[s % 2].wait()
            prep(s + 2)
            p[s].wait()
    p[_K - 2].wait()
    p[_K - 1].wait()


def kernel(x, table):
    xt = x.T.reshape(_N).astype(jnp.int32)
    out = _emb_lookup(xt, table)
    return out.reshape(_F, _B, _EMBED_DIM).transpose(1, 0, 2)


# speedup vs baseline: 2.7426x; 1.6420x over previous
import functools

import jax
import jax.numpy as jnp
from jax import lax
from jax.experimental import pallas as pl
from jax.experimental.pallas import tpu as pltpu
from jax.experimental.pallas import tpu_sc as plsc

_FIELD_DIM = 100000
_EMBED_DIM = 32
_B = 4096
_F = 26
_N = _B * _F
_NW = 32
_CHUNK = _N // _NW
_LANES = 16

_K = 26
_S = _CHUNK // _K

_mesh = plsc.VectorSubcoreMesh(core_axis_name="c", subcore_axis_name="s")


@functools.partial(
    pl.kernel,
    mesh=_mesh,
    out_type=jax.ShapeDtypeStruct((_N, _EMBED_DIM), jnp.float32),
    scratch_types=[
        pltpu.VMEM((_S,), jnp.int32),
        pltpu.VMEM((_S,), jnp.int32),
        pltpu.VMEM((_S,), jnp.int32),
        pltpu.VMEM((_S,), jnp.int32),
        pltpu.VMEM((_S, _EMBED_DIM), jnp.float32),
        pltpu.VMEM((_S, _EMBED_DIM), jnp.float32),
        pltpu.SemaphoreType.DMA((2,)),
        pltpu.SemaphoreType.DMA((2,)),
        pltpu.SemaphoreType.DMA((2,)),
        pltpu.SemaphoreType.DMA((2,)),
    ],
    compiler_params=pltpu.CompilerParams(needs_layout_passes=False),
)
def _emb_lookup(xt_hbm, tab_hbm, out_hbm, x_v0, x_v1, p_v0, p_v1,
                rows_v0, rows_v1, xsem, ssem, gsem, psem):
    wid = lax.axis_index("s") * 2 + lax.axis_index("c")
    base = wid * _CHUNK
    lane = lax.iota(jnp.int32, _LANES)
    x_v = [x_v0, x_v1]
    p_v = [p_v0, p_v1]
    rows_v = [rows_v0, rows_v1]

    def start_xload(s):
        return pltpu.async_copy(
            xt_hbm.at[pl.ds(base + s * _S, _S)], x_v[s % 2], xsem.at[s % 2])

    def prep(s):
        b = s % 2

        def body(t, carry):
            sl = pl.ds(t * _LANES, _LANES)
            pos = (base + s * _S) + t * _LANES + lane
            p_v[b][sl] = x_v[b][sl] + (pos >> 12) * _FIELD_DIM
            return carry

        lax.fori_loop(0, _S // _LANES, body, 0)

    def gather(s):
        b = s % 2

        def body(i, carry):
            grp = p_v[b][pl.ds((i >> 4) * _LANES, _LANES)]
            r = lax.reduce_max(
                jnp.where(lane == (i & 15), grp, 0), axes=(0,))
            pltpu.async_copy(
                tab_hbm.at[0, pl.ds(r, 1)], rows_v[b].at[pl.ds(i, 1)],
                gsem.at[b])
            return carry

        lax.fori_loop(0, _S, body, 0)

    def drain_gather(s):
        b = s % 2
        pltpu.make_async_copy(
            tab_hbm.at[0, pl.ds(0, _S)], rows_v[b], gsem.at[b]).wait()

    def start_put(s):
        b = s % 2
        return pltpu.async_copy(
            rows_v[b], out_hbm.at[pl.ds(base + s * _S, _S)], psem.at[b])

    xs = [start_xload(0), start_xload(1)]
    p = [None] * _K
    xs[0].wait()
    prep(0)
    xs[1].wait()
    prep(1)
    gather(0)
    for s in range(_K):
        drain_gather(s)
        p[s] = start_put(s)
        if s + 1 < _K:
            gather(s + 1)
        if s + 2 < _K:
            xs[s % 2] = start_xload(s + 2)
            xs[s % 2].wait()
            prep(s + 2)
            p[s].wait()
    p[_K - 2].wait()
    p[_K - 1].wait()


def kernel(x, table):
    xt = x.T.reshape(_N).astype(jnp.int32)
    out = _emb_lookup(xt, table.reshape(1, table.shape[0], _EMBED_DIM))
    return out.reshape(_F, _B, _EMBED_DIM).transpose(1, 0, 2)


# speedup vs baseline: 2.9743x; 1.0845x over previous
import functools

import jax
import jax.numpy as jnp
from jax import lax
from jax.experimental import pallas as pl
from jax.experimental.pallas import tpu as pltpu
from jax.experimental.pallas import tpu_sc as plsc

_FIELD_DIM = 100000
_EMBED_DIM = 32
_B = 4096
_F = 26
_N = _B * _F
_NW = 32
_CHUNK = _N // _NW
_LANES = 16

_K = 13
_S = _CHUNK // _K

_mesh = plsc.VectorSubcoreMesh(core_axis_name="c", subcore_axis_name="s")


@functools.partial(
    pl.kernel,
    mesh=_mesh,
    out_type=jax.ShapeDtypeStruct((_N, _EMBED_DIM), jnp.float32),
    scratch_types=[
        pltpu.VMEM((_S,), jnp.int32),
        pltpu.VMEM((_S,), jnp.int32),
        pltpu.VMEM((_S,), jnp.int32),
        pltpu.VMEM((_S,), jnp.int32),
        pltpu.VMEM((_S, _EMBED_DIM), jnp.float32),
        pltpu.VMEM((_S, _EMBED_DIM), jnp.float32),
        pltpu.SemaphoreType.DMA((2,)),
        pltpu.SemaphoreType.DMA((2,)),
        pltpu.SemaphoreType.DMA((2,)),
        pltpu.SemaphoreType.DMA((2,)),
    ],
    compiler_params=pltpu.CompilerParams(needs_layout_passes=False),
)
def _emb_lookup(xt_hbm, tab_hbm, out_hbm, x_v0, x_v1, p_v0, p_v1,
                rows_v0, rows_v1, xsem, ssem, gsem, psem):
    wid = lax.axis_index("s") * 2 + lax.axis_index("c")
    base = wid * _CHUNK
    lane = lax.iota(jnp.int32, _LANES)
    x_v = [x_v0, x_v1]
    p_v = [p_v0, p_v1]
    rows_v = [rows_v0, rows_v1]

    def start_xload(s):
        return pltpu.async_copy(
            xt_hbm.at[pl.ds(base + s * _S, _S)], x_v[s % 2], xsem.at[s % 2])

    def prep(s):
        b = s % 2

        def body(t, carry):
            sl = pl.ds(t * _LANES, _LANES)
            pos = (base + s * _S) + t * _LANES + lane
            p_v[b][sl] = x_v[b][sl] + (pos >> 12) * _FIELD_DIM
            return carry

        lax.fori_loop(0, _S // _LANES, body, 0)

    def gather(s):
        b = s % 2

        def body(t, carry):
            grp = p_v[b][pl.ds(t * _LANES, _LANES)]
            for j in range(_LANES):
                i = t * _LANES + j
                r = lax.reduce_max(
                    jnp.where(lane == j, grp, 0), axes=(0,))
                pltpu.async_copy(
                    tab_hbm.at[0, pl.ds(r, 1)], rows_v[b].at[pl.ds(i, 1)],
                    gsem.at[b])
            return carry

        lax.fori_loop(0, _S // _LANES, body, 0)

    def drain_gather(s):
        b = s % 2
        pltpu.make_async_copy(
            tab_hbm.at[0, pl.ds(0, _S)], rows_v[b], gsem.at[b]).wait()

    def start_put(s):
        b = s % 2
        return pltpu.async_copy(
            rows_v[b], out_hbm.at[pl.ds(base + s * _S, _S)], psem.at[b])

    xs = [start_xload(0), start_xload(1)]
    p = [None] * _K
    xs[0].wait()
    prep(0)
    xs[1].wait()
    prep(1)
    gather(0)
    for s in range(_K):
        drain_gather(s)
        p[s] = start_put(s)
        if s + 1 < _K:
            gather(s + 1)
        if s + 2 < _K:
            xs[s % 2] = start_xload(s + 2)
            xs[s % 2].wait()
            prep(s + 2)
            p[s].wait()
    p[_K - 2].wait()
    p[_K - 1].wait()


def kernel(x, table):
    xt = x.T.reshape(_N).astype(jnp.int32)
    out = _emb_lookup(xt, table.reshape(1, table.shape[0], _EMBED_DIM))
    return out.reshape(_F, _B, _EMBED_DIM).transpose(1, 0, 2)
